# baseline (speedup 1.0000x reference)
import functools

import jax
import jax.numpy as jnp
from jax import lax
from jax.experimental import pallas as pl
from jax.experimental.pallas import tpu as pltpu
from jax.experimental.pallas import tpu_sc as plsc

N = 10000
E = 320000
H = 128
G = 256
VOCAB_PAD = 128

NBLK = 10
BN = N // NBLK
NSC = 2
NSUB = 16
NTILES = NSC * NSUB
EPT = E // NTILES
ECH = 104
NCH = 97
EPT_P = NCH * ECH
RPTB = 624
TAIL = N - NSUB * RPTB


def _make_edge_agg():
    mesh = plsc.VectorSubcoreMesh(core_axis_name="c", subcore_axis_name="s")

    @functools.partial(
        pl.kernel,
        mesh=mesh,
        out_type=jax.ShapeDtypeStruct((NSC, N, H), jnp.float32),
        scratch_types=[
            pltpu.VMEM_SHARED((N + 8, H), jnp.float32),
            pltpu.VMEM((EPT_P,), jnp.int32),
            pltpu.VMEM((NCH, ECH), jnp.int32),
            pltpu.VMEM((ECH, H), jnp.float32),
            pltpu.VMEM((ECH, H), jnp.float32),
            pltpu.SemaphoreType.DMA,
            pltpu.SemaphoreType.DMA,
        ],
    )
    def edge_agg(h_hbm, src_hbm, dst_hbm, zero_hbm, out_hbm,
                 acc_sh, src_v, dst_v, rows0_v, rows1_v, sem0, sem1):
        c = lax.axis_index("c")
        s = lax.axis_index("s")
        wid = c * NSUB + s
        r0 = s * RPTB
        pltpu.sync_copy(zero_hbm.at[pl.ds(r0, RPTB)], acc_sh.at[pl.ds(r0, RPTB)])

        @pl.when(s == NSUB - 1)
        def _():
            pltpu.sync_copy(zero_hbm.at[pl.ds(NSUB * RPTB, TAIL)],
                            acc_sh.at[pl.ds(NSUB * RPTB, TAIL)])

        pltpu.sync_copy(src_hbm.at[wid], src_v)
        pltpu.sync_copy(dst_hbm.at[wid], dst_v)
        plsc.subcore_barrier()

        SG0 = 56
        SG1 = ECH - SG0

        def gather(j, buf, sem):
            pltpu.async_copy(
                h_hbm.at[src_v.at[pl.ds(j * ECH, SG0)]],
                buf.at[pl.ds(0, SG0)], sem)
            pltpu.async_copy(
                h_hbm.at[src_v.at[pl.ds(j * ECH + SG0, SG1)]],
                buf.at[pl.ds(SG0, SG1)], sem)

        gather(0, rows0_v, sem0)
        gather(1, rows1_v, sem1)

        def body(k, carry):
            j0 = 2 * k
            pltpu.make_async_copy(h_hbm.at[src_v.at[pl.ds(0, ECH)]], rows0_v, sem0).wait()
            pltpu.sync_copy(rows0_v, acc_sh.at[dst_v.at[j0]], add=True)
            gather(jnp.minimum(j0 + 2, NCH - 1), rows0_v, sem0)
            pltpu.make_async_copy(h_hbm.at[src_v.at[pl.ds(0, ECH)]], rows1_v, sem1).wait()
            pltpu.sync_copy(rows1_v, acc_sh.at[dst_v.at[j0 + 1]], add=True)
            gather(jnp.minimum(j0 + 3, NCH - 1), rows1_v, sem1)
            return carry

        lax.fori_loop(0, NCH // 2, body, 0)
        pltpu.make_async_copy(h_hbm.at[src_v.at[pl.ds(0, ECH)]], rows0_v, sem0).wait()
        pltpu.sync_copy(rows0_v, acc_sh.at[dst_v.at[NCH - 1]], add=True)
        pltpu.make_async_copy(h_hbm.at[src_v.at[pl.ds(0, ECH)]], rows1_v, sem1).wait()
        plsc.subcore_barrier()
        pltpu.sync_copy(acc_sh.at[pl.ds(r0, RPTB)],
                        out_hbm.at[c, pl.ds(r0, RPTB)])

        @pl.when(s == NSUB - 1)
        def _():
            pltpu.sync_copy(acc_sh.at[pl.ds(NSUB * RPTB, TAIL)],
                            out_hbm.at[c, pl.ds(NSUB * RPTB, TAIL)])

    return edge_agg


_EDGE_AGG = _make_edge_agg()


def _embed_body(x_ref, tab_ref, out_ref):
    xb = x_ref[0, 0, :]
    oh = (xb[:, None] ==
          lax.broadcasted_iota(jnp.int32, (BN, VOCAB_PAD), 1)).astype(jnp.float32)
    out_ref[...] = jnp.dot(oh, tab_ref[...], preferred_element_type=jnp.float32)


def _embed(x, tab_padded):
    x3 = x.reshape(NBLK, 1, BN)
    return pl.pallas_call(
        _embed_body,
        grid=(NBLK,),
        in_specs=[
            pl.BlockSpec((1, 1, BN), lambda i: (i, 0, 0)),
            pl.BlockSpec((VOCAB_PAD, H), lambda i: (0, 0)),
        ],
        out_specs=pl.BlockSpec((BN, H), lambda i: (i, 0)),
        out_shape=jax.ShapeDtypeStruct((N, H), jnp.float32),
    )(x3, tab_padded)


def _mlp_body(eps_ref, h_ref, agg_ref, w1_ref, b1_ref, w2_ref, b2_ref,
              z2_ref, sums_ref, acc_ref):
    i = pl.program_id(0)
    z = (1.0 + eps_ref[0, 0]) * h_ref[...] + agg_ref[0] + agg_ref[1]
    z1 = jnp.maximum(
        jnp.dot(z, w1_ref[...], preferred_element_type=jnp.float32,
                precision=lax.Precision.HIGHEST) + b1_ref[...],
        0.0)
    z2 = jnp.dot(z1, w2_ref[...], preferred_element_type=jnp.float32,
                 precision=lax.Precision.HIGHEST) + b2_ref[...]
    z2_ref[...] = z2

    @pl.when(i == 0)
    def _():
        acc_ref[...] = jnp.zeros_like(acc_ref)

    acc_ref[0:1, :] += jnp.sum(z2, axis=0, keepdims=True)
    acc_ref[1:2, :] += jnp.sum(z2 * z2, axis=0, keepdims=True)

    @pl.when(i == NBLK - 1)
    def _():
        sums_ref[...] = acc_ref[...]


def _mlp(eps2, h, agg, w1, b1r, w2, b2r):
    return pl.pallas_call(
        _mlp_body,
        grid=(NBLK,),
        in_specs=[
            pl.BlockSpec(memory_space=pltpu.SMEM),
            pl.BlockSpec((BN, H), lambda i: (i, 0)),
            pl.BlockSpec((NSC, BN, H), lambda i: (0, i, 0)),
            pl.BlockSpec((H, H), lambda i: (0, 0)),
            pl.BlockSpec((1, H), lambda i: (0, 0)),
            pl.BlockSpec((H, H), lambda i: (0, 0)),
            pl.BlockSpec((1, H), lambda i: (0, 0)),
        ],
        out_specs=[
            pl.BlockSpec((BN, H), lambda i: (i, 0)),
            pl.BlockSpec((8, H), lambda i: (0, 0)),
        ],
        out_shape=[
            jax.ShapeDtypeStruct((N, H), jnp.float32),
            jax.ShapeDtypeStruct((8, H), jnp.float32),
        ],
        scratch_shapes=[pltpu.VMEM((8, H), jnp.float32)],
    )(eps2, h, agg, w1, b1r, w2, b2r)


def _bn_body(z2_ref, h_ref, sums_ref, gamma_ref, beta_ref, out_ref):
    mu = sums_ref[0:1, :] * (1.0 / N)
    ex2 = sums_ref[1:2, :] * (1.0 / N)
    var = ex2 - mu * mu
    inv = lax.rsqrt(var + 1e-5)
    out_ref[...] = ((z2_ref[...] - mu) * (inv * gamma_ref[...])
                    + beta_ref[...] + h_ref[...])


def _bn(z2, h, sums, gammar, betar):
    return pl.pallas_call(
        _bn_body,
        grid=(NBLK,),
        in_specs=[
            pl.BlockSpec((BN, H), lambda i: (i, 0)),
            pl.BlockSpec((BN, H), lambda i: (i, 0)),
            pl.BlockSpec((8, H), lambda i: (0, 0)),
            pl.BlockSpec((1, H), lambda i: (0, 0)),
            pl.BlockSpec((1, H), lambda i: (0, 0)),
        ],
        out_specs=pl.BlockSpec((BN, H), lambda i: (i, 0)),
        out_shape=jax.ShapeDtypeStruct((N, H), jnp.float32),
    )(z2, h, sums, gammar, betar)


def _pool_body(b_ref, h_ref, out_ref, acc_ref):
    i = pl.program_id(0)
    b = b_ref[0, 0, :]
    oht = (lax.broadcasted_iota(jnp.int32, (G, BN), 0) ==
           b[None, :]).astype(jnp.float32)
    part = jnp.dot(oht, h_ref[...], preferred_element_type=jnp.float32)

    @pl.when(i == 0)
    def _():
        acc_ref[...] = jnp.zeros_like(acc_ref)

    acc_ref[...] += part

    @pl.when(i == NBLK - 1)
    def _():
        out_ref[...] = acc_ref[...]


def _pool(batch, h):
    b3 = batch.reshape(NBLK, 1, BN)
    return pl.pallas_call(
        _pool_body,
        grid=(NBLK,),
        in_specs=[
            pl.BlockSpec((1, 1, BN), lambda i: (i, 0, 0)),
            pl.BlockSpec((BN, H), lambda i: (i, 0)),
        ],
        out_specs=pl.BlockSpec((G, H), lambda i: (0, 0)),
        out_shape=jax.ShapeDtypeStruct((G, H), jnp.float32),
        scratch_shapes=[pltpu.VMEM((G, H), jnp.float32)],
    )(b3, h)


def kernel(x, edge_index, edge_attr, batch, params):
    del edge_attr
    table = params["atom_table"]
    tab_padded = jnp.concatenate(
        [table, jnp.zeros((VOCAB_PAD - table.shape[0], H), jnp.float32)], axis=0)
    pad = EPT_P - EPT
    src = jnp.concatenate(
        [edge_index[0].reshape(NTILES, EPT),
         jnp.zeros((NTILES, pad), edge_index.dtype)], axis=1)
    dst = jnp.concatenate(
        [edge_index[1].reshape(NTILES, EPT),
         jnp.full((NTILES, pad), N, edge_index.dtype)],
        axis=1).reshape(NTILES, NCH, ECH)
    zeros_nh = jnp.zeros((N, H), jnp.float32)

    h = _embed(x, tab_padded)
    for p in params["layers"]:
        agg = _EDGE_AGG(h, src, dst, zeros_nh)
        eps2 = p["eps"].reshape(1, 1)
        z2, sums = _mlp(eps2, h, agg, p["W1"], p["b1"].reshape(1, H),
                        p["W2"], p["b2"].reshape(1, H))
        h = _bn(z2, h, sums, p["gamma"].reshape(1, H), p["beta"].reshape(1, H))
    return _pool(batch, h)


# speedup vs baseline: 1.0468x; 1.0468x over previous
import functools

import jax
import jax.numpy as jnp
from jax import lax
from jax.experimental import pallas as pl
from jax.experimental.pallas import tpu as pltpu
from jax.experimental.pallas import tpu_sc as plsc

N = 10000
E = 320000
H = 128
G = 256
VOCAB_PAD = 128

NBLK = 10
BN = N // NBLK
NSC = 2
NSUB = 16
NTILES = NSC * NSUB
EPT = E // NTILES
ECH = 96
NCH = 105
EPT_P = NCH * ECH
RPTB = 624
TAIL = N - NSUB * RPTB


def _make_edge_agg():
    mesh = plsc.VectorSubcoreMesh(core_axis_name="c", subcore_axis_name="s")

    @functools.partial(
        pl.kernel,
        mesh=mesh,
        out_type=jax.ShapeDtypeStruct((NSC, N, H), jnp.float32),
        scratch_types=[
            pltpu.VMEM_SHARED((N + 8, H), jnp.float32),
            pltpu.VMEM((EPT_P,), jnp.int32),
            pltpu.VMEM((NCH, ECH), jnp.int32),
            pltpu.VMEM((ECH, H), jnp.float32),
            pltpu.VMEM((ECH, H), jnp.float32),
            pltpu.SemaphoreType.DMA,
            pltpu.SemaphoreType.DMA,
        ],
    )
    def edge_agg(h_hbm, src_hbm, dst_hbm, zero_hbm, out_hbm,
                 acc_sh, src_v, dst_v, rows0_v, rows1_v, sem0, sem1):
        c = lax.axis_index("c")
        s = lax.axis_index("s")
        wid = c * NSUB + s
        r0 = s * RPTB
        pltpu.sync_copy(zero_hbm.at[pl.ds(r0, RPTB)], acc_sh.at[pl.ds(r0, RPTB)])

        @pl.when(s == NSUB - 1)
        def _():
            pltpu.sync_copy(zero_hbm.at[pl.ds(NSUB * RPTB, TAIL)],
                            acc_sh.at[pl.ds(NSUB * RPTB, TAIL)])

        pltpu.sync_copy(src_hbm.at[wid], src_v)
        pltpu.sync_copy(dst_hbm.at[wid], dst_v)
        plsc.subcore_barrier()

        SG0 = ECH // 2
        SG1 = ECH - SG0

        def gather(j, buf, sem):
            pltpu.async_copy(
                h_hbm.at[src_v.at[pl.ds(j * ECH, SG0)]],
                buf.at[pl.ds(0, SG0)], sem)
            pltpu.async_copy(
                h_hbm.at[src_v.at[pl.ds(j * ECH + SG0, SG1)]],
                buf.at[pl.ds(SG0, SG1)], sem)

        gather(0, rows0_v, sem0)
        gather(1, rows1_v, sem1)

        def body(k, carry):
            j0 = 2 * k
            pltpu.make_async_copy(h_hbm.at[src_v.at[pl.ds(0, ECH)]], rows0_v, sem0).wait()
            pltpu.sync_copy(rows0_v, acc_sh.at[dst_v.at[j0]], add=True)
            gather(jnp.minimum(j0 + 2, NCH - 1), rows0_v, sem0)
            pltpu.make_async_copy(h_hbm.at[src_v.at[pl.ds(0, ECH)]], rows1_v, sem1).wait()
            pltpu.sync_copy(rows1_v, acc_sh.at[dst_v.at[j0 + 1]], add=True)
            gather(jnp.minimum(j0 + 3, NCH - 1), rows1_v, sem1)
            return carry

        lax.fori_loop(0, NCH // 2, body, 0)
        pltpu.make_async_copy(h_hbm.at[src_v.at[pl.ds(0, ECH)]], rows0_v, sem0).wait()
        pltpu.sync_copy(rows0_v, acc_sh.at[dst_v.at[NCH - 1]], add=True)
        pltpu.make_async_copy(h_hbm.at[src_v.at[pl.ds(0, ECH)]], rows1_v, sem1).wait()
        plsc.subcore_barrier()
        pltpu.sync_copy(acc_sh.at[pl.ds(r0, RPTB)],
                        out_hbm.at[c, pl.ds(r0, RPTB)])

        @pl.when(s == NSUB - 1)
        def _():
            pltpu.sync_copy(acc_sh.at[pl.ds(NSUB * RPTB, TAIL)],
                            out_hbm.at[c, pl.ds(NSUB * RPTB, TAIL)])

    return edge_agg


_EDGE_AGG = _make_edge_agg()


def _embed_body(x_ref, tab_ref, out_ref):
    xb = x_ref[0, 0, :]
    oh = (xb[:, None] ==
          lax.broadcasted_iota(jnp.int32, (BN, VOCAB_PAD), 1)).astype(jnp.float32)
    out_ref[...] = jnp.dot(oh, tab_ref[...], preferred_element_type=jnp.float32)


def _embed(x, tab_padded):
    x3 = x.reshape(NBLK, 1, BN)
    return pl.pallas_call(
        _embed_body,
        grid=(NBLK,),
        in_specs=[
            pl.BlockSpec((1, 1, BN), lambda i: (i, 0, 0)),
            pl.BlockSpec((VOCAB_PAD, H), lambda i: (0, 0)),
        ],
        out_specs=pl.BlockSpec((BN, H), lambda i: (i, 0)),
        out_shape=jax.ShapeDtypeStruct((N, H), jnp.float32),
    )(x3, tab_padded)


def _mlp_body(eps_ref, h_ref, agg_ref, w1_ref, b1_ref, w2_ref, b2_ref,
              z2_ref, sums_ref, acc_ref):
    i = pl.program_id(0)
    z = (1.0 + eps_ref[0, 0]) * h_ref[...] + agg_ref[0] + agg_ref[1]
    z1 = jnp.maximum(
        jnp.dot(z, w1_ref[...], preferred_element_type=jnp.float32,
                precision=lax.Precision.HIGHEST) + b1_ref[...],
        0.0)
    z2 = jnp.dot(z1, w2_ref[...], preferred_element_type=jnp.float32,
                 precision=lax.Precision.HIGHEST) + b2_ref[...]
    z2_ref[...] = z2

    @pl.when(i == 0)
    def _():
        acc_ref[...] = jnp.zeros_like(acc_ref)

    acc_ref[0:1, :] += jnp.sum(z2, axis=0, keepdims=True)
    acc_ref[1:2, :] += jnp.sum(z2 * z2, axis=0, keepdims=True)

    @pl.when(i == NBLK - 1)
    def _():
        sums_ref[...] = acc_ref[...]


def _mlp(eps2, h, agg, w1, b1r, w2, b2r):
    return pl.pallas_call(
        _mlp_body,
        grid=(NBLK,),
        in_specs=[
            pl.BlockSpec(memory_space=pltpu.SMEM),
            pl.BlockSpec((BN, H), lambda i: (i, 0)),
            pl.BlockSpec((NSC, BN, H), lambda i: (0, i, 0)),
            pl.BlockSpec((H, H), lambda i: (0, 0)),
            pl.BlockSpec((1, H), lambda i: (0, 0)),
            pl.BlockSpec((H, H), lambda i: (0, 0)),
            pl.BlockSpec((1, H), lambda i: (0, 0)),
        ],
        out_specs=[
            pl.BlockSpec((BN, H), lambda i: (i, 0)),
            pl.BlockSpec((8, H), lambda i: (0, 0)),
        ],
        out_shape=[
            jax.ShapeDtypeStruct((N, H), jnp.float32),
            jax.ShapeDtypeStruct((8, H), jnp.float32),
        ],
        scratch_shapes=[pltpu.VMEM((8, H), jnp.float32)],
    )(eps2, h, agg, w1, b1r, w2, b2r)


def _bn_body(z2_ref, h_ref, sums_ref, gamma_ref, beta_ref, out_ref):
    mu = sums_ref[0:1, :] * (1.0 / N)
    ex2 = sums_ref[1:2, :] * (1.0 / N)
    var = ex2 - mu * mu
    inv = lax.rsqrt(var + 1e-5)
    out_ref[...] = ((z2_ref[...] - mu) * (inv * gamma_ref[...])
                    + beta_ref[...] + h_ref[...])


def _bn(z2, h, sums, gammar, betar):
    return pl.pallas_call(
        _bn_body,
        grid=(NBLK,),
        in_specs=[
            pl.BlockSpec((BN, H), lambda i: (i, 0)),
            pl.BlockSpec((BN, H), lambda i: (i, 0)),
            pl.BlockSpec((8, H), lambda i: (0, 0)),
            pl.BlockSpec((1, H), lambda i: (0, 0)),
            pl.BlockSpec((1, H), lambda i: (0, 0)),
        ],
        out_specs=pl.BlockSpec((BN, H), lambda i: (i, 0)),
        out_shape=jax.ShapeDtypeStruct((N, H), jnp.float32),
    )(z2, h, sums, gammar, betar)


def _pool_body(b_ref, h_ref, out_ref, acc_ref):
    i = pl.program_id(0)
    b = b_ref[0, 0, :]
    oht = (lax.broadcasted_iota(jnp.int32, (G, BN), 0) ==
           b[None, :]).astype(jnp.float32)
    part = jnp.dot(oht, h_ref[...], preferred_element_type=jnp.float32)

    @pl.when(i == 0)
    def _():
        acc_ref[...] = jnp.zeros_like(acc_ref)

    acc_ref[...] += part

    @pl.when(i == NBLK - 1)
    def _():
        out_ref[...] = acc_ref[...]


def _pool(batch, h):
    b3 = batch.reshape(NBLK, 1, BN)
    return pl.pallas_call(
        _pool_body,
        grid=(NBLK,),
        in_specs=[
            pl.BlockSpec((1, 1, BN), lambda i: (i, 0, 0)),
            pl.BlockSpec((BN, H), lambda i: (i, 0)),
        ],
        out_specs=pl.BlockSpec((G, H), lambda i: (0, 0)),
        out_shape=jax.ShapeDtypeStruct((G, H), jnp.float32),
        scratch_shapes=[pltpu.VMEM((G, H), jnp.float32)],
    )(b3, h)


def kernel(x, edge_index, edge_attr, batch, params):
    del edge_attr
    table = params["atom_table"]
    tab_padded = jnp.concatenate(
        [table, jnp.zeros((VOCAB_PAD - table.shape[0], H), jnp.float32)], axis=0)
    pad = EPT_P - EPT
    src = jnp.concatenate(
        [edge_index[0].reshape(NTILES, EPT),
         jnp.zeros((NTILES, pad), edge_index.dtype)], axis=1)
    dst = jnp.concatenate(
        [edge_index[1].reshape(NTILES, EPT),
         jnp.full((NTILES, pad), N, edge_index.dtype)],
        axis=1).reshape(NTILES, NCH, ECH)
    zeros_nh = jnp.zeros((N, H), jnp.float32)

    h = _embed(x, tab_padded)
    for p in params["layers"]:
        agg = _EDGE_AGG(h, src, dst, zeros_nh)
        eps2 = p["eps"].reshape(1, 1)
        z2, sums = _mlp(eps2, h, agg, p["W1"], p["b1"].reshape(1, H),
                        p["W2"], p["b2"].reshape(1, H))
        h = _bn(z2, h, sums, p["gamma"].reshape(1, H), p["beta"].reshape(1, H))
    return _pool(batch, h)


# speedup vs baseline: 1.0475x; 1.0006x over previous
import functools

import jax
import jax.numpy as jnp
from jax import lax
from jax.experimental import pallas as pl
from jax.experimental.pallas import tpu as pltpu
from jax.experimental.pallas import tpu_sc as plsc

N = 10000
E = 320000
H = 128
G = 256
VOCAB_PAD = 128

NBLK = 10
BN = N // NBLK
NSC = 2
NSUB = 16
NTILES = NSC * NSUB
EPT = E // NTILES
ECH = 96
NCH = 105
EPT_P = NCH * ECH
RPTB = 624
TAIL = N - NSUB * RPTB


def _make_edge_agg():
    mesh = plsc.VectorSubcoreMesh(core_axis_name="c", subcore_axis_name="s")

    @functools.partial(
        pl.kernel,
        mesh=mesh,
        out_type=jax.ShapeDtypeStruct((NSC, N, H), jnp.float32),
        scratch_types=[
            pltpu.VMEM_SHARED((N + 8, H), jnp.float32),
            pltpu.VMEM((EPT_P,), jnp.int32),
            pltpu.VMEM((NCH, ECH), jnp.int32),
            pltpu.VMEM((ECH, H), jnp.float32),
            pltpu.VMEM((ECH, H), jnp.float32),
            pltpu.SemaphoreType.DMA,
            pltpu.SemaphoreType.DMA,
        ],
    )
    def edge_agg(h_hbm, src_hbm, dst_hbm, zero_hbm, out_hbm,
                 acc_sh, src_v, dst_v, rows0_v, rows1_v, sem0, sem1):
        c = lax.axis_index("c")
        s = lax.axis_index("s")
        wid = c * NSUB + s
        r0 = s * RPTB
        pltpu.sync_copy(zero_hbm.at[pl.ds(r0, RPTB)], acc_sh.at[pl.ds(r0, RPTB)])

        @pl.when(s == NSUB - 1)
        def _():
            pltpu.sync_copy(zero_hbm.at[pl.ds(NSUB * RPTB, TAIL)],
                            acc_sh.at[pl.ds(NSUB * RPTB, TAIL)])

        pltpu.sync_copy(src_hbm.at[wid], src_v)
        pltpu.sync_copy(dst_hbm.at[wid], dst_v)
        plsc.subcore_barrier()

        SG0 = ECH // 2
        SG1 = ECH - SG0

        def gather(j, buf, sem):
            pltpu.async_copy(
                h_hbm.at[src_v.at[pl.ds(j * ECH, SG0)]],
                buf.at[pl.ds(0, SG0)], sem)
            pltpu.async_copy(
                h_hbm.at[src_v.at[pl.ds(j * ECH + SG0, SG1)]],
                buf.at[pl.ds(SG0, SG1)], sem)

        gather(0, rows0_v, sem0)
        gather(1, rows1_v, sem1)

        def body(k, carry):
            j0 = 2 * k
            pltpu.make_async_copy(h_hbm.at[src_v.at[pl.ds(0, ECH)]], rows0_v, sem0).wait()
            pltpu.sync_copy(rows0_v, acc_sh.at[dst_v.at[j0]], add=True)
            gather(jnp.minimum(j0 + 2, NCH - 1), rows0_v, sem0)
            pltpu.make_async_copy(h_hbm.at[src_v.at[pl.ds(0, ECH)]], rows1_v, sem1).wait()
            pltpu.sync_copy(rows1_v, acc_sh.at[dst_v.at[j0 + 1]], add=True)
            gather(jnp.minimum(j0 + 3, NCH - 1), rows1_v, sem1)
            return carry

        lax.fori_loop(0, NCH // 2, body, 0)
        pltpu.make_async_copy(h_hbm.at[src_v.at[pl.ds(0, ECH)]], rows0_v, sem0).wait()
        pltpu.sync_copy(rows0_v, acc_sh.at[dst_v.at[NCH - 1]], add=True)
        pltpu.make_async_copy(h_hbm.at[src_v.at[pl.ds(0, ECH)]], rows1_v, sem1).wait()
        plsc.subcore_barrier()
        pltpu.sync_copy(acc_sh.at[pl.ds(r0, RPTB)],
                        out_hbm.at[c, pl.ds(r0, RPTB)])

        @pl.when(s == NSUB - 1)
        def _():
            pltpu.sync_copy(acc_sh.at[pl.ds(NSUB * RPTB, TAIL)],
                            out_hbm.at[c, pl.ds(NSUB * RPTB, TAIL)])

    return edge_agg


_EDGE_AGG = _make_edge_agg()


def _embed_body(x_ref, tab_ref, out_ref):
    xb = x_ref[0, 0, :]
    oh = (xb[:, None] ==
          lax.broadcasted_iota(jnp.int32, (BN, VOCAB_PAD), 1)).astype(jnp.float32)
    out_ref[...] = jnp.dot(oh, tab_ref[...], preferred_element_type=jnp.float32)


def _embed(x, tab_padded):
    x3 = x.reshape(NBLK, 1, BN)
    return pl.pallas_call(
        _embed_body,
        grid=(NBLK,),
        in_specs=[
            pl.BlockSpec((1, 1, BN), lambda i: (i, 0, 0)),
            pl.BlockSpec((VOCAB_PAD, H), lambda i: (0, 0)),
        ],
        out_specs=pl.BlockSpec((BN, H), lambda i: (i, 0)),
        out_shape=jax.ShapeDtypeStruct((N, H), jnp.float32),
    )(x3, tab_padded)


def _mlp_body(eps_ref, h_ref, agg_ref, w1_ref, b1_ref, w2_ref, b2_ref,
              z2_ref, sums_ref, acc_ref):
    i = pl.program_id(0)
    z = (1.0 + eps_ref[0, 0]) * h_ref[...] + agg_ref[0] + agg_ref[1]
    z1 = jnp.maximum(
        jnp.dot(z, w1_ref[...], preferred_element_type=jnp.float32,
                precision=lax.Precision.HIGHEST) + b1_ref[...],
        0.0)
    z2 = jnp.dot(z1, w2_ref[...], preferred_element_type=jnp.float32,
                 precision=lax.Precision.HIGHEST) + b2_ref[...]
    z2_ref[...] = z2

    @pl.when(i == 0)
    def _():
        acc_ref[...] = jnp.zeros_like(acc_ref)

    acc_ref[0:1, :] += jnp.sum(z2, axis=0, keepdims=True)
    acc_ref[1:2, :] += jnp.sum(z2 * z2, axis=0, keepdims=True)

    @pl.when(i == NBLK - 1)
    def _():
        sums_ref[...] = acc_ref[...]


def _mlp(eps2, h, agg, w1, b1r, w2, b2r):
    return pl.pallas_call(
        _mlp_body,
        grid=(NBLK,),
        in_specs=[
            pl.BlockSpec(memory_space=pltpu.SMEM),
            pl.BlockSpec((BN, H), lambda i: (i, 0)),
            pl.BlockSpec((NSC, BN, H), lambda i: (0, i, 0)),
            pl.BlockSpec((H, H), lambda i: (0, 0)),
            pl.BlockSpec((1, H), lambda i: (0, 0)),
            pl.BlockSpec((H, H), lambda i: (0, 0)),
            pl.BlockSpec((1, H), lambda i: (0, 0)),
        ],
        out_specs=[
            pl.BlockSpec((BN, H), lambda i: (i, 0)),
            pl.BlockSpec((8, H), lambda i: (0, 0)),
        ],
        out_shape=[
            jax.ShapeDtypeStruct((N, H), jnp.float32),
            jax.ShapeDtypeStruct((8, H), jnp.float32),
        ],
        scratch_shapes=[pltpu.VMEM((8, H), jnp.float32)],
    )(eps2, h, agg, w1, b1r, w2, b2r)


def _bn_body(z2_ref, h_ref, sums_ref, gamma_ref, beta_ref, out_ref):
    mu = sums_ref[0:1, :] * (1.0 / N)
    ex2 = sums_ref[1:2, :] * (1.0 / N)
    var = ex2 - mu * mu
    inv = lax.rsqrt(var + 1e-5)
    out_ref[...] = ((z2_ref[...] - mu) * (inv * gamma_ref[...])
                    + beta_ref[...] + h_ref[...])


def _bn(z2, h, sums, gammar, betar):
    return pl.pallas_call(
        _bn_body,
        grid=(NBLK,),
        in_specs=[
            pl.BlockSpec((BN, H), lambda i: (i, 0)),
            pl.BlockSpec((BN, H), lambda i: (i, 0)),
            pl.BlockSpec((8, H), lambda i: (0, 0)),
            pl.BlockSpec((1, H), lambda i: (0, 0)),
            pl.BlockSpec((1, H), lambda i: (0, 0)),
        ],
        out_specs=pl.BlockSpec((BN, H), lambda i: (i, 0)),
        out_shape=jax.ShapeDtypeStruct((N, H), jnp.float32),
    )(z2, h, sums, gammar, betar)


def _pool_body(b_ref, h_ref, out_ref, acc_ref):
    i = pl.program_id(0)
    b = b_ref[0, 0, :]
    oht = (lax.broadcasted_iota(jnp.int32, (G, BN), 0) ==
           b[None, :]).astype(jnp.float32)
    part = jnp.dot(oht, h_ref[...], preferred_element_type=jnp.float32)

    @pl.when(i == 0)
    def _():
        acc_ref[...] = jnp.zeros_like(acc_ref)

    acc_ref[...] += part

    @pl.when(i == NBLK - 1)
    def _():
        out_ref[...] = acc_ref[...]


def _pool(batch, h):
    b3 = batch.reshape(NBLK, 1, BN)
    return pl.pallas_call(
        _pool_body,
        grid=(NBLK,),
        in_specs=[
            pl.BlockSpec((1, 1, BN), lambda i: (i, 0, 0)),
            pl.BlockSpec((BN, H), lambda i: (i, 0)),
        ],
        out_specs=pl.BlockSpec((G, H), lambda i: (0, 0)),
        out_shape=jax.ShapeDtypeStruct((G, H), jnp.float32),
        scratch_shapes=[pltpu.VMEM((G, H), jnp.float32)],
    )(b3, h)


def kernel(x, edge_index, edge_attr, batch, params):
    del edge_attr
    table = params["atom_table"]
    tab_padded = jnp.concatenate(
        [table, jnp.zeros((VOCAB_PAD - table.shape[0], H), jnp.float32)], axis=0)
    pad = EPT_P - EPT
    src = jnp.concatenate(
        [edge_index[0].reshape(NTILES, EPT),
         jnp.zeros((NTILES, pad), edge_index.dtype)], axis=1)
    dst_pad = (N + jnp.arange(pad, dtype=edge_index.dtype) % 8)[None, :]
    dst = jnp.concatenate(
        [edge_index[1].reshape(NTILES, EPT),
         jnp.broadcast_to(dst_pad, (NTILES, pad))],
        axis=1).reshape(NTILES, NCH, ECH)
    zeros_nh = jnp.zeros((N, H), jnp.float32)

    h = _embed(x, tab_padded)
    for p in params["layers"]:
        agg = _EDGE_AGG(h, src, dst, zeros_nh)
        eps2 = p["eps"].reshape(1, 1)
        z2, sums = _mlp(eps2, h, agg, p["W1"], p["b1"].reshape(1, H),
                        p["W2"], p["b2"].reshape(1, H))
        h = _bn(z2, h, sums, p["gamma"].reshape(1, H), p["beta"].reshape(1, H))
    return _pool(batch, h)


# speedup vs baseline: 2.1503x; 2.0529x over previous
import functools

import jax
import jax.numpy as jnp
from jax import lax
from jax.experimental import pallas as pl
from jax.experimental.pallas import tpu as pltpu
from jax.experimental.pallas import tpu_sc as plsc

N = 10000
E = 320000
H = 128
G = 256
VOCAB_PAD = 128

NBLK = 10
BN = N // NBLK
NSC = 2
NSUB = 16
NTILES = NSC * NSUB
EPT = E // NTILES
ECH = 80
NCH = EPT // ECH
RPTB = 624
TAIL = N - NSUB * RPTB


def _make_edge_agg():
    mesh = plsc.VectorSubcoreMesh(core_axis_name="c", subcore_axis_name="s")

    @functools.partial(
        pl.kernel,
        mesh=mesh,
        out_type=jax.ShapeDtypeStruct((NSC, N, H), jnp.float32),
        scratch_types=[
            pltpu.VMEM_SHARED((N, H), jnp.float32),
            pltpu.VMEM((EPT,), jnp.int32),
            pltpu.VMEM((NCH, ECH), jnp.int32),
            pltpu.VMEM((ECH, H), jnp.float32),
            pltpu.VMEM((ECH, H), jnp.float32),
            pltpu.SemaphoreType.DMA,
            pltpu.SemaphoreType.DMA,
        ],
    )
    def edge_agg(h_hbm, src_hbm, dst_hbm, zero_hbm, out_hbm,
                 acc_sh, src_v, dst_v, rows0_v, rows1_v, sem0, sem1):
        c = lax.axis_index("c")
        s = lax.axis_index("s")
        wid = c * NSUB + s
        r0 = s * RPTB
        pltpu.sync_copy(zero_hbm.at[pl.ds(r0, RPTB)], acc_sh.at[pl.ds(r0, RPTB)])

        @pl.when(s == NSUB - 1)
        def _():
            pltpu.sync_copy(zero_hbm.at[pl.ds(NSUB * RPTB, TAIL)],
                            acc_sh.at[pl.ds(NSUB * RPTB, TAIL)])

        pltpu.sync_copy(src_hbm.at[wid], src_v)
        pltpu.sync_copy(dst_hbm.at[wid], dst_v)
        plsc.subcore_barrier()

        HECH = ECH // 2

        def gather(j, buf, sem):
            pltpu.async_copy(
                h_hbm.at[src_v.at[pl.ds(j * ECH, HECH)]],
                buf.at[pl.ds(0, HECH)], sem)
            pltpu.async_copy(
                h_hbm.at[src_v.at[pl.ds(j * ECH + HECH, HECH)]],
                buf.at[pl.ds(HECH, HECH)], sem)

        gather(0, rows0_v, sem0)
        gather(1, rows1_v, sem1)

        def body(k, carry):
            j0 = 2 * k
            pltpu.make_async_copy(h_hbm.at[src_v.at[pl.ds(0, ECH)]], rows0_v, sem0).wait()
            pltpu.sync_copy(rows0_v, acc_sh.at[dst_v.at[j0]], add=True)
            gather(jnp.minimum(j0 + 2, NCH - 1), rows0_v, sem0)
            pltpu.make_async_copy(h_hbm.at[src_v.at[pl.ds(0, ECH)]], rows1_v, sem1).wait()
            pltpu.sync_copy(rows1_v, acc_sh.at[dst_v.at[j0 + 1]], add=True)
            gather(jnp.minimum(j0 + 3, NCH - 1), rows1_v, sem1)
            return carry

        lax.fori_loop(0, NCH // 2, body, 0)
        pltpu.make_async_copy(h_hbm.at[src_v.at[pl.ds(0, ECH)]], rows0_v, sem0).wait()
        pltpu.sync_copy(rows0_v, acc_sh.at[dst_v.at[NCH - 1]], add=True)
        pltpu.make_async_copy(h_hbm.at[src_v.at[pl.ds(0, ECH)]], rows1_v, sem1).wait()
        plsc.subcore_barrier()
        pltpu.sync_copy(acc_sh.at[pl.ds(r0, RPTB)],
                        out_hbm.at[c, pl.ds(r0, RPTB)])

        @pl.when(s == NSUB - 1)
        def _():
            pltpu.sync_copy(acc_sh.at[pl.ds(NSUB * RPTB, TAIL)],
                            out_hbm.at[c, pl.ds(NSUB * RPTB, TAIL)])

    return edge_agg


_EDGE_AGG = _make_edge_agg()


def _embed_body(x_ref, tab_ref, out_ref):
    xb = x_ref[0, 0, :]
    oh = (xb[:, None] ==
          lax.broadcasted_iota(jnp.int32, (BN, VOCAB_PAD), 1)).astype(jnp.float32)
    out_ref[...] = jnp.dot(oh, tab_ref[...], preferred_element_type=jnp.float32)


def _embed(x, tab_padded):
    x3 = x.reshape(NBLK, 1, BN)
    return pl.pallas_call(
        _embed_body,
        grid=(NBLK,),
        in_specs=[
            pl.BlockSpec((1, 1, BN), lambda i: (i, 0, 0)),
            pl.BlockSpec((VOCAB_PAD, H), lambda i: (0, 0)),
        ],
        out_specs=pl.BlockSpec((BN, H), lambda i: (i, 0)),
        out_shape=jax.ShapeDtypeStruct((N, H), jnp.float32),
    )(x3, tab_padded)


def _layer_body(eps_ref, h_ref, agg_ref, w1_ref, b1_ref, w2_ref, b2_ref,
                gamma_ref, beta_ref, out_ref, z2_acc, stat_acc):
    p = pl.program_id(0)
    i = pl.program_id(1)

    @pl.when(p == 0)
    def _():
        z = (1.0 + eps_ref[0, 0]) * h_ref[...] + agg_ref[0] + agg_ref[1]
        z1 = jnp.maximum(
            jnp.dot(z, w1_ref[...], preferred_element_type=jnp.float32,
                    precision=lax.Precision.HIGHEST) + b1_ref[...],
            0.0)
        z2 = jnp.dot(z1, w2_ref[...], preferred_element_type=jnp.float32,
                     precision=lax.Precision.HIGHEST) + b2_ref[...]
        z2_acc[pl.ds(i * BN, BN), :] = z2

        @pl.when(i == 0)
        def _():
            stat_acc[...] = jnp.zeros_like(stat_acc)

        stat_acc[0:1, :] += jnp.sum(z2, axis=0, keepdims=True)
        stat_acc[1:2, :] += jnp.sum(z2 * z2, axis=0, keepdims=True)

    @pl.when(p == 1)
    def _():
        mu = stat_acc[0:1, :] * (1.0 / N)
        ex2 = stat_acc[1:2, :] * (1.0 / N)
        var = ex2 - mu * mu
        inv = lax.rsqrt(var + 1e-5)
        out_ref[...] = ((z2_acc[pl.ds(i * BN, BN), :] - mu)
                        * (inv * gamma_ref[...])
                        + beta_ref[...] + h_ref[...])


def _layer(eps2, h, agg, w1, b1r, w2, b2r, gammar, betar):
    return pl.pallas_call(
        _layer_body,
        grid=(2, NBLK),
        in_specs=[
            pl.BlockSpec(memory_space=pltpu.SMEM),
            pl.BlockSpec((BN, H), lambda p, i: (i, 0)),
            pl.BlockSpec((NSC, BN, H), lambda p, i: (0, jnp.where(p == 0, i, 0), 0)),
            pl.BlockSpec((H, H), lambda p, i: (0, 0)),
            pl.BlockSpec((1, H), lambda p, i: (0, 0)),
            pl.BlockSpec((H, H), lambda p, i: (0, 0)),
            pl.BlockSpec((1, H), lambda p, i: (0, 0)),
            pl.BlockSpec((1, H), lambda p, i: (0, 0)),
            pl.BlockSpec((1, H), lambda p, i: (0, 0)),
        ],
        out_specs=pl.BlockSpec((BN, H), lambda p, i: (jnp.where(p == 1, i, 0), 0)),
        out_shape=jax.ShapeDtypeStruct((N, H), jnp.float32),
        scratch_shapes=[
            pltpu.VMEM((N, H), jnp.float32),
            pltpu.VMEM((8, H), jnp.float32),
        ],
    )(eps2, h, agg, w1, b1r, w2, b2r, gammar, betar)


def _pool_body(b_ref, h_ref, out_ref, acc_ref):
    i = pl.program_id(0)
    b = b_ref[0, 0, :]
    oht = (lax.broadcasted_iota(jnp.int32, (G, BN), 0) ==
           b[None, :]).astype(jnp.float32)
    part = jnp.dot(oht, h_ref[...], preferred_element_type=jnp.float32)

    @pl.when(i == 0)
    def _():
        acc_ref[...] = jnp.zeros_like(acc_ref)

    acc_ref[...] += part

    @pl.when(i == NBLK - 1)
    def _():
        out_ref[...] = acc_ref[...]


def _pool(batch, h):
    b3 = batch.reshape(NBLK, 1, BN)
    return pl.pallas_call(
        _pool_body,
        grid=(NBLK,),
        in_specs=[
            pl.BlockSpec((1, 1, BN), lambda i: (i, 0, 0)),
            pl.BlockSpec((BN, H), lambda i: (i, 0)),
        ],
        out_specs=pl.BlockSpec((G, H), lambda i: (0, 0)),
        out_shape=jax.ShapeDtypeStruct((G, H), jnp.float32),
        scratch_shapes=[pltpu.VMEM((G, H), jnp.float32)],
    )(b3, h)


def kernel(x, edge_index, edge_attr, batch, params):
    del edge_attr
    table = params["atom_table"]
    tab_padded = jnp.concatenate(
        [table, jnp.zeros((VOCAB_PAD - table.shape[0], H), jnp.float32)], axis=0)
    src = edge_index[0].reshape(NTILES, EPT)
    dst = edge_index[1].reshape(NTILES, NCH, ECH)
    zeros_nh = jnp.zeros((N, H), jnp.float32)

    h = _embed(x, tab_padded)
    for p in params["layers"]:
        agg = _EDGE_AGG(h, src, dst, zeros_nh)
        eps2 = p["eps"].reshape(1, 1)
        h = _layer(eps2, h, agg, p["W1"], p["b1"].reshape(1, H),
                   p["W2"], p["b2"].reshape(1, H),
                   p["gamma"].reshape(1, H), p["beta"].reshape(1, H))
    return _pool(batch, h)


# speedup vs baseline: 2.4868x; 1.1565x over previous
import functools

import jax
import jax.numpy as jnp
from jax import lax
from jax.experimental import pallas as pl
from jax.experimental.pallas import tpu as pltpu
from jax.experimental.pallas import tpu_sc as plsc

N = 10000
E = 320000
H = 128
G = 256
VOCAB_PAD = 128

NBLK = 10
BN = N // NBLK
NSC = 2
NSUB = 16
NTILES = NSC * NSUB
EPT = E // NTILES
ECH = 80
NCH = EPT // ECH
RPTB = 624
TAIL = N - NSUB * RPTB


def _make_edge_agg():
    mesh = plsc.VectorSubcoreMesh(core_axis_name="c", subcore_axis_name="s")

    @functools.partial(
        pl.kernel,
        mesh=mesh,
        out_type=jax.ShapeDtypeStruct((NSC, N, H), jnp.float32),
        scratch_types=[
            pltpu.VMEM_SHARED((N, H), jnp.float32),
            pltpu.VMEM((EPT,), jnp.int32),
            pltpu.VMEM((NCH, ECH), jnp.int32),
            pltpu.VMEM((ECH, H), jnp.float32),
            pltpu.VMEM((ECH, H), jnp.float32),
            pltpu.SemaphoreType.DMA,
            pltpu.SemaphoreType.DMA,
        ],
    )
    def edge_agg(h_hbm, src_hbm, dst_hbm, zero_hbm, out_hbm,
                 acc_sh, src_v, dst_v, rows0_v, rows1_v, sem0, sem1):
        c = lax.axis_index("c")
        s = lax.axis_index("s")
        wid = c * NSUB + s
        r0 = s * RPTB
        pltpu.sync_copy(zero_hbm.at[pl.ds(r0, RPTB)], acc_sh.at[pl.ds(r0, RPTB)])

        @pl.when(s == NSUB - 1)
        def _():
            pltpu.sync_copy(zero_hbm.at[pl.ds(NSUB * RPTB, TAIL)],
                            acc_sh.at[pl.ds(NSUB * RPTB, TAIL)])

        pltpu.sync_copy(src_hbm.at[wid], src_v)
        pltpu.sync_copy(dst_hbm.at[wid], dst_v)
        plsc.subcore_barrier()

        HECH = ECH // 2

        def gather(j, buf, sem):
            pltpu.async_copy(
                h_hbm.at[src_v.at[pl.ds(j * ECH, HECH)]],
                buf.at[pl.ds(0, HECH)], sem)
            pltpu.async_copy(
                h_hbm.at[src_v.at[pl.ds(j * ECH + HECH, HECH)]],
                buf.at[pl.ds(HECH, HECH)], sem)

        gather(0, rows0_v, sem0)
        gather(1, rows1_v, sem1)

        def body(k, carry):
            j0 = 2 * k
            pltpu.make_async_copy(h_hbm.at[src_v.at[pl.ds(0, ECH)]], rows0_v, sem0).wait()
            pltpu.sync_copy(rows0_v, acc_sh.at[dst_v.at[j0]], add=True)
            gather(jnp.minimum(j0 + 2, NCH - 1), rows0_v, sem0)
            pltpu.make_async_copy(h_hbm.at[src_v.at[pl.ds(0, ECH)]], rows1_v, sem1).wait()
            pltpu.sync_copy(rows1_v, acc_sh.at[dst_v.at[j0 + 1]], add=True)
            gather(jnp.minimum(j0 + 3, NCH - 1), rows1_v, sem1)
            return carry

        lax.fori_loop(0, NCH // 2, body, 0)
        pltpu.make_async_copy(h_hbm.at[src_v.at[pl.ds(0, ECH)]], rows0_v, sem0).wait()
        pltpu.sync_copy(rows0_v, acc_sh.at[dst_v.at[NCH - 1]], add=True)
        pltpu.make_async_copy(h_hbm.at[src_v.at[pl.ds(0, ECH)]], rows1_v, sem1).wait()
        plsc.subcore_barrier()
        pltpu.sync_copy(acc_sh.at[pl.ds(r0, RPTB)],
                        out_hbm.at[c, pl.ds(r0, RPTB)])

        @pl.when(s == NSUB - 1)
        def _():
            pltpu.sync_copy(acc_sh.at[pl.ds(NSUB * RPTB, TAIL)],
                            out_hbm.at[c, pl.ds(NSUB * RPTB, TAIL)])

    return edge_agg


_EDGE_AGG = _make_edge_agg()


def _embed_body(x_ref, tab_ref, out_ref):
    xb = x_ref[0, 0, :]
    oh = (xb[:, None] ==
          lax.broadcasted_iota(jnp.int32, (BN, VOCAB_PAD), 1)).astype(jnp.float32)
    out_ref[...] = jnp.dot(oh, tab_ref[...], preferred_element_type=jnp.float32)


def _embed(x, tab_padded):
    x3 = x.reshape(NBLK, 1, BN)
    return pl.pallas_call(
        _embed_body,
        grid=(NBLK,),
        in_specs=[
            pl.BlockSpec((1, 1, BN), lambda i: (i, 0, 0)),
            pl.BlockSpec((VOCAB_PAD, H), lambda i: (0, 0)),
        ],
        out_specs=pl.BlockSpec((BN, H), lambda i: (i, 0)),
        out_shape=jax.ShapeDtypeStruct((N, H), jnp.float32),
    )(x3, tab_padded)


def _layer_body(eps_ref, h_ref, agg_ref, w1_ref, b1_ref, w2_ref, b2_ref,
                gamma_ref, beta_ref, out_ref, z2_acc, stat_acc):
    p = pl.program_id(0)
    i = pl.program_id(1)

    @pl.when(p == 0)
    def _():
        z = (1.0 + eps_ref[0, 0]) * h_ref[...] + agg_ref[0] + agg_ref[1]
        z1 = jnp.maximum(
            jnp.dot(z, w1_ref[...], preferred_element_type=jnp.float32) + b1_ref[...],
            0.0)
        z2 = jnp.dot(z1, w2_ref[...], preferred_element_type=jnp.float32) + b2_ref[...]
        z2_acc[pl.ds(i * BN, BN), :] = z2

        @pl.when(i == 0)
        def _():
            stat_acc[...] = jnp.zeros_like(stat_acc)

        stat_acc[0:1, :] += jnp.sum(z2, axis=0, keepdims=True)
        stat_acc[1:2, :] += jnp.sum(z2 * z2, axis=0, keepdims=True)

    @pl.when(p == 1)
    def _():
        mu = stat_acc[0:1, :] * (1.0 / N)
        ex2 = stat_acc[1:2, :] * (1.0 / N)
        var = ex2 - mu * mu
        inv = lax.rsqrt(var + 1e-5)
        out_ref[...] = ((z2_acc[pl.ds(i * BN, BN), :] - mu)
                        * (inv * gamma_ref[...])
                        + beta_ref[...] + h_ref[...])


def _layer(eps2, h, agg, w1, b1r, w2, b2r, gammar, betar):
    return pl.pallas_call(
        _layer_body,
        grid=(2, NBLK),
        in_specs=[
            pl.BlockSpec(memory_space=pltpu.SMEM),
            pl.BlockSpec((BN, H), lambda p, i: (i, 0)),
            pl.BlockSpec((NSC, BN, H), lambda p, i: (0, jnp.where(p == 0, i, 0), 0)),
            pl.BlockSpec((H, H), lambda p, i: (0, 0)),
            pl.BlockSpec((1, H), lambda p, i: (0, 0)),
            pl.BlockSpec((H, H), lambda p, i: (0, 0)),
            pl.BlockSpec((1, H), lambda p, i: (0, 0)),
            pl.BlockSpec((1, H), lambda p, i: (0, 0)),
            pl.BlockSpec((1, H), lambda p, i: (0, 0)),
        ],
        out_specs=pl.BlockSpec((BN, H), lambda p, i: (jnp.where(p == 1, i, 0), 0)),
        out_shape=jax.ShapeDtypeStruct((N, H), jnp.float32),
        scratch_shapes=[
            pltpu.VMEM((N, H), jnp.float32),
            pltpu.VMEM((8, H), jnp.float32),
        ],
    )(eps2, h, agg, w1, b1r, w2, b2r, gammar, betar)


def _pool_body(b_ref, h_ref, out_ref, acc_ref):
    i = pl.program_id(0)
    b = b_ref[0, 0, :]
    oht = (lax.broadcasted_iota(jnp.int32, (G, BN), 0) ==
           b[None, :]).astype(jnp.float32)
    part = jnp.dot(oht, h_ref[...], preferred_element_type=jnp.float32)

    @pl.when(i == 0)
    def _():
        acc_ref[...] = jnp.zeros_like(acc_ref)

    acc_ref[...] += part

    @pl.when(i == NBLK - 1)
    def _():
        out_ref[...] = acc_ref[...]


def _pool(batch, h):
    b3 = batch.reshape(NBLK, 1, BN)
    return pl.pallas_call(
        _pool_body,
        grid=(NBLK,),
        in_specs=[
            pl.BlockSpec((1, 1, BN), lambda i: (i, 0, 0)),
            pl.BlockSpec((BN, H), lambda i: (i, 0)),
        ],
        out_specs=pl.BlockSpec((G, H), lambda i: (0, 0)),
        out_shape=jax.ShapeDtypeStruct((G, H), jnp.float32),
        scratch_shapes=[pltpu.VMEM((G, H), jnp.float32)],
    )(b3, h)


def kernel(x, edge_index, edge_attr, batch, params):
    del edge_attr
    table = params["atom_table"]
    tab_padded = jnp.concatenate(
        [table, jnp.zeros((VOCAB_PAD - table.shape[0], H), jnp.float32)], axis=0)
    src = edge_index[0].reshape(NTILES, EPT)
    dst = edge_index[1].reshape(NTILES, NCH, ECH)
    zeros_nh = jnp.zeros((N, H), jnp.float32)

    h = _embed(x, tab_padded)
    for p in params["layers"]:
        agg = _EDGE_AGG(h, src, dst, zeros_nh)
        eps2 = p["eps"].reshape(1, 1)
        h = _layer(eps2, h, agg, p["W1"], p["b1"].reshape(1, H),
                   p["W2"], p["b2"].reshape(1, H),
                   p["gamma"].reshape(1, H), p["beta"].reshape(1, H))
    return _pool(batch, h)


# speedup vs baseline: 2.4974x; 1.0043x over previous
import functools

import jax
import jax.numpy as jnp
from jax import lax
from jax.experimental import pallas as pl
from jax.experimental.pallas import tpu as pltpu
from jax.experimental.pallas import tpu_sc as plsc

N = 10000
E = 320000
H = 128
G = 256
VOCAB_PAD = 128

NBLK = 10
BN = N // NBLK
NSC = 2
NSUB = 16
NTILES = NSC * NSUB
EPT = E // NTILES
ECH = 80
NCH = EPT // ECH
RPTB = 624
TAIL = N - NSUB * RPTB


def _make_edge_agg():
    mesh = plsc.VectorSubcoreMesh(core_axis_name="c", subcore_axis_name="s")

    @functools.partial(
        pl.kernel,
        mesh=mesh,
        out_type=jax.ShapeDtypeStruct((NSC, N, H), jnp.float32),
        scratch_types=[
            pltpu.VMEM_SHARED((N, H), jnp.float32),
            pltpu.VMEM((EPT,), jnp.int32),
            pltpu.VMEM((NCH, ECH), jnp.int32),
            pltpu.VMEM((ECH, H), jnp.float32),
            pltpu.VMEM((ECH, H), jnp.float32),
            pltpu.SemaphoreType.DMA,
            pltpu.SemaphoreType.DMA,
        ],
    )
    def edge_agg(h_hbm, src_hbm, dst_hbm, zero_hbm, out_hbm,
                 acc_sh, src_v, dst_v, rows0_v, rows1_v, sem0, sem1):
        c = lax.axis_index("c")
        s = lax.axis_index("s")
        wid = c * NSUB + s
        r0 = s * RPTB
        pltpu.sync_copy(zero_hbm.at[pl.ds(r0, RPTB)], acc_sh.at[pl.ds(r0, RPTB)])

        @pl.when(s == NSUB - 1)
        def _():
            pltpu.sync_copy(zero_hbm.at[pl.ds(NSUB * RPTB, TAIL)],
                            acc_sh.at[pl.ds(NSUB * RPTB, TAIL)])

        pltpu.sync_copy(src_hbm.at[wid], src_v)
        pltpu.sync_copy(dst_hbm.at[wid], dst_v)
        plsc.subcore_barrier()

        HECH = ECH // 2

        def gather(j, buf, sem):
            pltpu.async_copy(
                h_hbm.at[src_v.at[pl.ds(j * ECH, HECH)]],
                buf.at[pl.ds(0, HECH)], sem)
            pltpu.async_copy(
                h_hbm.at[src_v.at[pl.ds(j * ECH + HECH, HECH)]],
                buf.at[pl.ds(HECH, HECH)], sem)

        gather(0, rows0_v, sem0)
        gather(1, rows1_v, sem1)

        def body(k, carry):
            j0 = 2 * k
            pltpu.make_async_copy(h_hbm.at[src_v.at[pl.ds(0, ECH)]], rows0_v, sem0).wait()
            pltpu.sync_copy(rows0_v, acc_sh.at[dst_v.at[j0]], add=True)
            gather(jnp.minimum(j0 + 2, NCH - 1), rows0_v, sem0)
            pltpu.make_async_copy(h_hbm.at[src_v.at[pl.ds(0, ECH)]], rows1_v, sem1).wait()
            pltpu.sync_copy(rows1_v, acc_sh.at[dst_v.at[j0 + 1]], add=True)
            gather(jnp.minimum(j0 + 3, NCH - 1), rows1_v, sem1)
            return carry

        lax.fori_loop(0, NCH // 2, body, 0)
        pltpu.make_async_copy(h_hbm.at[src_v.at[pl.ds(0, ECH)]], rows0_v, sem0).wait()
        pltpu.sync_copy(rows0_v, acc_sh.at[dst_v.at[NCH - 1]], add=True)
        pltpu.make_async_copy(h_hbm.at[src_v.at[pl.ds(0, ECH)]], rows1_v, sem1).wait()
        plsc.subcore_barrier()
        pltpu.sync_copy(acc_sh.at[pl.ds(r0, RPTB)],
                        out_hbm.at[c, pl.ds(r0, RPTB)])

        @pl.when(s == NSUB - 1)
        def _():
            pltpu.sync_copy(acc_sh.at[pl.ds(NSUB * RPTB, TAIL)],
                            out_hbm.at[c, pl.ds(NSUB * RPTB, TAIL)])

    return edge_agg


_EDGE_AGG = _make_edge_agg()


def _embed_body(x_ref, tab_ref, out_ref):
    xb = x_ref[0, 0, :]
    oh = (xb[:, None] ==
          lax.broadcasted_iota(jnp.int32, (BN, VOCAB_PAD), 1)).astype(jnp.float32)
    out_ref[...] = jnp.dot(oh, tab_ref[...], preferred_element_type=jnp.float32)


def _embed(x, tab_padded):
    x3 = x.reshape(NBLK, 1, BN)
    return pl.pallas_call(
        _embed_body,
        grid=(NBLK,),
        in_specs=[
            pl.BlockSpec((1, 1, BN), lambda i: (i, 0, 0)),
            pl.BlockSpec((VOCAB_PAD, H), lambda i: (0, 0)),
        ],
        out_specs=pl.BlockSpec((BN, H), lambda i: (i, 0)),
        out_shape=jax.ShapeDtypeStruct((N, H), jnp.float32),
    )(x3, tab_padded)


def _layer_body(eps_ref, h_ref, agg_ref, w1_ref, b1_ref, w2_ref, b2_ref,
                gamma_ref, beta_ref, out_ref, z2_acc, stat_acc):
    p = pl.program_id(0)
    i = pl.program_id(1)

    @pl.when(p == 0)
    def _():
        z = (1.0 + eps_ref[0, 0]) * h_ref[...] + agg_ref[0] + agg_ref[1]
        z1 = jnp.maximum(
            jnp.dot(z, w1_ref[...], preferred_element_type=jnp.float32) + b1_ref[...],
            0.0)
        z2 = jnp.dot(z1, w2_ref[...], preferred_element_type=jnp.float32) + b2_ref[...]
        z2_acc[pl.ds(i * BN, BN), :] = z2

        @pl.when(i == 0)
        def _():
            stat_acc[...] = jnp.zeros_like(stat_acc)

        stat_acc[0:1, :] += jnp.sum(z2, axis=0, keepdims=True)
        stat_acc[1:2, :] += jnp.sum(z2 * z2, axis=0, keepdims=True)

    @pl.when(p == 1)
    def _():
        mu = stat_acc[0:1, :] * (1.0 / N)
        ex2 = stat_acc[1:2, :] * (1.0 / N)
        var = ex2 - mu * mu
        inv = lax.rsqrt(var + 1e-5)
        out_ref[...] = ((z2_acc[pl.ds(i * BN, BN), :] - mu)
                        * (inv * gamma_ref[...])
                        + beta_ref[...] + h_ref[...])


def _layer(eps2, h, agg, w1, b1r, w2, b2r, gammar, betar):
    return pl.pallas_call(
        _layer_body,
        grid=(2, NBLK),
        in_specs=[
            pl.BlockSpec(memory_space=pltpu.SMEM),
            pl.BlockSpec((BN, H), lambda p, i: (i, 0)),
            pl.BlockSpec((NSC, BN, H), lambda p, i: (0, jnp.where(p == 0, i, 0), 0)),
            pl.BlockSpec((H, H), lambda p, i: (0, 0)),
            pl.BlockSpec((1, H), lambda p, i: (0, 0)),
            pl.BlockSpec((H, H), lambda p, i: (0, 0)),
            pl.BlockSpec((1, H), lambda p, i: (0, 0)),
            pl.BlockSpec((1, H), lambda p, i: (0, 0)),
            pl.BlockSpec((1, H), lambda p, i: (0, 0)),
        ],
        out_specs=pl.BlockSpec((BN, H), lambda p, i: (jnp.where(p == 1, i, 0), 0)),
        out_shape=jax.ShapeDtypeStruct((N, H), jnp.float32),
        scratch_shapes=[
            pltpu.VMEM((N, H), jnp.float32),
            pltpu.VMEM((8, H), jnp.float32),
        ],
    )(eps2, h, agg, w1, b1r, w2, b2r, gammar, betar)


def _last_layer_body(eps_ref, h_ref, agg_ref, w1_ref, b1_ref, w2_ref, b2_ref,
                     gamma_ref, beta_ref, b_ref, out_ref, z2_acc, stat_acc,
                     pool_acc):
    p = pl.program_id(0)
    i = pl.program_id(1)

    @pl.when(p == 0)
    def _():
        z = (1.0 + eps_ref[0, 0]) * h_ref[...] + agg_ref[0] + agg_ref[1]
        z1 = jnp.maximum(
            jnp.dot(z, w1_ref[...], preferred_element_type=jnp.float32)
            + b1_ref[...], 0.0)
        z2 = (jnp.dot(z1, w2_ref[...], preferred_element_type=jnp.float32)
              + b2_ref[...])
        z2_acc[pl.ds(i * BN, BN), :] = z2

        @pl.when(i == 0)
        def _():
            stat_acc[...] = jnp.zeros_like(stat_acc)

        stat_acc[0:1, :] += jnp.sum(z2, axis=0, keepdims=True)
        stat_acc[1:2, :] += jnp.sum(z2 * z2, axis=0, keepdims=True)

    @pl.when(p == 1)
    def _():
        mu = stat_acc[0:1, :] * (1.0 / N)
        ex2 = stat_acc[1:2, :] * (1.0 / N)
        var = ex2 - mu * mu
        inv = lax.rsqrt(var + 1e-5)
        z2_acc[pl.ds(i * BN, BN), :] = (
            (z2_acc[pl.ds(i * BN, BN), :] - mu) * (inv * gamma_ref[...])
            + beta_ref[...] + h_ref[...])

    @pl.when(p == 2)
    def _():
        b = b_ref[0, 0, :]
        oht = (lax.broadcasted_iota(jnp.int32, (G, BN), 0) ==
               b[None, :]).astype(jnp.float32)
        part = jnp.dot(oht, z2_acc[pl.ds(i * BN, BN), :],
                       preferred_element_type=jnp.float32)

        @pl.when(i == 0)
        def _():
            pool_acc[...] = jnp.zeros_like(pool_acc)

        pool_acc[...] += part

        @pl.when(i == NBLK - 1)
        def _():
            out_ref[...] = pool_acc[...]


def _last_layer(eps2, h, agg, w1, b1r, w2, b2r, gammar, betar, batch):
    b3 = batch.reshape(NBLK, 1, BN)
    return pl.pallas_call(
        _last_layer_body,
        grid=(3, NBLK),
        in_specs=[
            pl.BlockSpec(memory_space=pltpu.SMEM),
            pl.BlockSpec((BN, H), lambda p, i: (jnp.where(p == 2, 0, i), 0)),
            pl.BlockSpec((NSC, BN, H), lambda p, i: (0, jnp.where(p == 0, i, 0), 0)),
            pl.BlockSpec((H, H), lambda p, i: (0, 0)),
            pl.BlockSpec((1, H), lambda p, i: (0, 0)),
            pl.BlockSpec((H, H), lambda p, i: (0, 0)),
            pl.BlockSpec((1, H), lambda p, i: (0, 0)),
            pl.BlockSpec((1, H), lambda p, i: (0, 0)),
            pl.BlockSpec((1, H), lambda p, i: (0, 0)),
            pl.BlockSpec((1, 1, BN), lambda p, i: (jnp.where(p == 2, i, 0), 0, 0)),
        ],
        out_specs=pl.BlockSpec((G, H), lambda p, i: (0, 0)),
        out_shape=jax.ShapeDtypeStruct((G, H), jnp.float32),
        scratch_shapes=[
            pltpu.VMEM((N, H), jnp.float32),
            pltpu.VMEM((8, H), jnp.float32),
            pltpu.VMEM((G, H), jnp.float32),
        ],
    )(eps2, h, agg, w1, b1r, w2, b2r, gammar, betar, b3)


def kernel(x, edge_index, edge_attr, batch, params):
    del edge_attr
    table = params["atom_table"]
    tab_padded = jnp.concatenate(
        [table, jnp.zeros((VOCAB_PAD - table.shape[0], H), jnp.float32)], axis=0)
    src = edge_index[0].reshape(NTILES, EPT)
    dst = edge_index[1].reshape(NTILES, NCH, ECH)
    zeros_nh = jnp.zeros((N, H), jnp.float32)

    h = _embed(x, tab_padded)
    for li, p in enumerate(params["layers"]):
        agg = _EDGE_AGG(h, src, dst, zeros_nh)
        eps2 = p["eps"].reshape(1, 1)
        args = (eps2, h, agg, p["W1"], p["b1"].reshape(1, H),
                p["W2"], p["b2"].reshape(1, H),
                p["gamma"].reshape(1, H), p["beta"].reshape(1, H))
        if li == len(params["layers"]) - 1:
            return _last_layer(*args, batch)
        h = _layer(*args)


# speedup vs baseline: 2.4991x; 1.0006x over previous
import functools

import jax
import jax.numpy as jnp
from jax import lax
from jax.experimental import pallas as pl
from jax.experimental.pallas import tpu as pltpu
from jax.experimental.pallas import tpu_sc as plsc

N = 10000
E = 320000
H = 128
G = 256
VOCAB_PAD = 128

NBLK = 10
BN = N // NBLK
NSC = 2
NSUB = 16
NTILES = NSC * NSUB
EPT = E // NTILES
ECH = 80
NCH = EPT // ECH
RPTB = 624
TAIL = N - NSUB * RPTB


def _make_edge_agg():
    mesh = plsc.VectorSubcoreMesh(core_axis_name="c", subcore_axis_name="s")

    @functools.partial(
        pl.kernel,
        mesh=mesh,
        out_type=jax.ShapeDtypeStruct((NSC, N, H), jnp.float32),
        scratch_types=[
            pltpu.VMEM_SHARED((N, H), jnp.float32),
            pltpu.VMEM((EPT,), jnp.int32),
            pltpu.VMEM((NCH, ECH), jnp.int32),
            pltpu.VMEM((ECH, H), jnp.float32),
            pltpu.VMEM((ECH, H), jnp.float32),
            pltpu.SemaphoreType.DMA,
            pltpu.SemaphoreType.DMA,
        ],
    )
    def edge_agg(h_hbm, src_hbm, dst_hbm, zero_hbm, out_hbm,
                 acc_sh, src_v, dst_v, rows0_v, rows1_v, sem0, sem1):
        c = lax.axis_index("c")
        s = lax.axis_index("s")
        wid = c * NSUB + s
        r0 = s * RPTB
        pltpu.sync_copy(zero_hbm.at[pl.ds(r0, RPTB)], acc_sh.at[pl.ds(r0, RPTB)])

        @pl.when(s == NSUB - 1)
        def _():
            pltpu.sync_copy(zero_hbm.at[pl.ds(NSUB * RPTB, TAIL)],
                            acc_sh.at[pl.ds(NSUB * RPTB, TAIL)])

        pltpu.sync_copy(src_hbm.at[wid], src_v)
        pltpu.sync_copy(dst_hbm.at[wid], dst_v)
        plsc.subcore_barrier()

        def gather(j, buf, sem):
            off = 0
            for sz in (24, 16, 24, 16):
                pltpu.async_copy(
                    h_hbm.at[src_v.at[pl.ds(j * ECH + off, sz)]],
                    buf.at[pl.ds(off, sz)], sem)
                off += sz

        gather(0, rows0_v, sem0)
        gather(1, rows1_v, sem1)

        def body(k, carry):
            j0 = 2 * k
            pltpu.make_async_copy(h_hbm.at[src_v.at[pl.ds(0, ECH)]], rows0_v, sem0).wait()
            pltpu.sync_copy(rows0_v, acc_sh.at[dst_v.at[j0]], add=True)
            gather(jnp.minimum(j0 + 2, NCH - 1), rows0_v, sem0)
            pltpu.make_async_copy(h_hbm.at[src_v.at[pl.ds(0, ECH)]], rows1_v, sem1).wait()
            pltpu.sync_copy(rows1_v, acc_sh.at[dst_v.at[j0 + 1]], add=True)
            gather(jnp.minimum(j0 + 3, NCH - 1), rows1_v, sem1)
            return carry

        lax.fori_loop(0, NCH // 2, body, 0)
        pltpu.make_async_copy(h_hbm.at[src_v.at[pl.ds(0, ECH)]], rows0_v, sem0).wait()
        pltpu.sync_copy(rows0_v, acc_sh.at[dst_v.at[NCH - 1]], add=True)
        pltpu.make_async_copy(h_hbm.at[src_v.at[pl.ds(0, ECH)]], rows1_v, sem1).wait()
        plsc.subcore_barrier()
        pltpu.sync_copy(acc_sh.at[pl.ds(r0, RPTB)],
                        out_hbm.at[c, pl.ds(r0, RPTB)])

        @pl.when(s == NSUB - 1)
        def _():
            pltpu.sync_copy(acc_sh.at[pl.ds(NSUB * RPTB, TAIL)],
                            out_hbm.at[c, pl.ds(NSUB * RPTB, TAIL)])

    return edge_agg


_EDGE_AGG = _make_edge_agg()


def _embed_body(x_ref, tab_ref, out_ref):
    xb = x_ref[0, 0, :]
    oh = (xb[:, None] ==
          lax.broadcasted_iota(jnp.int32, (BN, VOCAB_PAD), 1)).astype(jnp.float32)
    out_ref[...] = jnp.dot(oh, tab_ref[...], preferred_element_type=jnp.float32)


def _embed(x, tab_padded):
    x3 = x.reshape(NBLK, 1, BN)
    return pl.pallas_call(
        _embed_body,
        grid=(NBLK,),
        in_specs=[
            pl.BlockSpec((1, 1, BN), lambda i: (i, 0, 0)),
            pl.BlockSpec((VOCAB_PAD, H), lambda i: (0, 0)),
        ],
        out_specs=pl.BlockSpec((BN, H), lambda i: (i, 0)),
        out_shape=jax.ShapeDtypeStruct((N, H), jnp.float32),
    )(x3, tab_padded)


def _layer_body(eps_ref, h_ref, agg_ref, w1_ref, b1_ref, w2_ref, b2_ref,
                gamma_ref, beta_ref, out_ref, z2_acc, stat_acc):
    p = pl.program_id(0)
    i = pl.program_id(1)

    @pl.when(p == 0)
    def _():
        z = (1.0 + eps_ref[0, 0]) * h_ref[...] + agg_ref[0] + agg_ref[1]
        z1 = jnp.maximum(
            jnp.dot(z, w1_ref[...], preferred_element_type=jnp.float32) + b1_ref[...],
            0.0)
        z2 = jnp.dot(z1, w2_ref[...], preferred_element_type=jnp.float32) + b2_ref[...]
        z2_acc[pl.ds(i * BN, BN), :] = z2

        @pl.when(i == 0)
        def _():
            stat_acc[...] = jnp.zeros_like(stat_acc)

        stat_acc[0:1, :] += jnp.sum(z2, axis=0, keepdims=True)
        stat_acc[1:2, :] += jnp.sum(z2 * z2, axis=0, keepdims=True)

    @pl.when(p == 1)
    def _():
        mu = stat_acc[0:1, :] * (1.0 / N)
        ex2 = stat_acc[1:2, :] * (1.0 / N)
        var = ex2 - mu * mu
        inv = lax.rsqrt(var + 1e-5)
        out_ref[...] = ((z2_acc[pl.ds(i * BN, BN), :] - mu)
                        * (inv * gamma_ref[...])
                        + beta_ref[...] + h_ref[...])


def _layer(eps2, h, agg, w1, b1r, w2, b2r, gammar, betar):
    return pl.pallas_call(
        _layer_body,
        grid=(2, NBLK),
        in_specs=[
            pl.BlockSpec(memory_space=pltpu.SMEM),
            pl.BlockSpec((BN, H), lambda p, i: (i, 0)),
            pl.BlockSpec((NSC, BN, H), lambda p, i: (0, jnp.where(p == 0, i, 0), 0)),
            pl.BlockSpec((H, H), lambda p, i: (0, 0)),
            pl.BlockSpec((1, H), lambda p, i: (0, 0)),
            pl.BlockSpec((H, H), lambda p, i: (0, 0)),
            pl.BlockSpec((1, H), lambda p, i: (0, 0)),
            pl.BlockSpec((1, H), lambda p, i: (0, 0)),
            pl.BlockSpec((1, H), lambda p, i: (0, 0)),
        ],
        out_specs=pl.BlockSpec((BN, H), lambda p, i: (jnp.where(p == 1, i, 0), 0)),
        out_shape=jax.ShapeDtypeStruct((N, H), jnp.float32),
        scratch_shapes=[
            pltpu.VMEM((N, H), jnp.float32),
            pltpu.VMEM((8, H), jnp.float32),
        ],
    )(eps2, h, agg, w1, b1r, w2, b2r, gammar, betar)


def _last_layer_body(eps_ref, h_ref, agg_ref, w1_ref, b1_ref, w2_ref, b2_ref,
                     gamma_ref, beta_ref, b_ref, out_ref, z2_acc, stat_acc,
                     pool_acc):
    p = pl.program_id(0)
    i = pl.program_id(1)

    @pl.when(p == 0)
    def _():
        z = (1.0 + eps_ref[0, 0]) * h_ref[...] + agg_ref[0] + agg_ref[1]
        z1 = jnp.maximum(
            jnp.dot(z, w1_ref[...], preferred_element_type=jnp.float32)
            + b1_ref[...], 0.0)
        z2 = (jnp.dot(z1, w2_ref[...], preferred_element_type=jnp.float32)
              + b2_ref[...])
        z2_acc[pl.ds(i * BN, BN), :] = z2

        @pl.when(i == 0)
        def _():
            stat_acc[...] = jnp.zeros_like(stat_acc)

        stat_acc[0:1, :] += jnp.sum(z2, axis=0, keepdims=True)
        stat_acc[1:2, :] += jnp.sum(z2 * z2, axis=0, keepdims=True)

    @pl.when(p == 1)
    def _():
        mu = stat_acc[0:1, :] * (1.0 / N)
        ex2 = stat_acc[1:2, :] * (1.0 / N)
        var = ex2 - mu * mu
        inv = lax.rsqrt(var + 1e-5)
        z2_acc[pl.ds(i * BN, BN), :] = (
            (z2_acc[pl.ds(i * BN, BN), :] - mu) * (inv * gamma_ref[...])
            + beta_ref[...] + h_ref[...])

    @pl.when(p == 2)
    def _():
        b = b_ref[0, 0, :]
        oht = (lax.broadcasted_iota(jnp.int32, (G, BN), 0) ==
               b[None, :]).astype(jnp.float32)
        part = jnp.dot(oht, z2_acc[pl.ds(i * BN, BN), :],
                       preferred_element_type=jnp.float32)

        @pl.when(i == 0)
        def _():
            pool_acc[...] = jnp.zeros_like(pool_acc)

        pool_acc[...] += part

        @pl.when(i == NBLK - 1)
        def _():
            out_ref[...] = pool_acc[...]


def _last_layer(eps2, h, agg, w1, b1r, w2, b2r, gammar, betar, batch):
    b3 = batch.reshape(NBLK, 1, BN)
    return pl.pallas_call(
        _last_layer_body,
        grid=(3, NBLK),
        in_specs=[
            pl.BlockSpec(memory_space=pltpu.SMEM),
            pl.BlockSpec((BN, H), lambda p, i: (jnp.where(p == 2, 0, i), 0)),
            pl.BlockSpec((NSC, BN, H), lambda p, i: (0, jnp.where(p == 0, i, 0), 0)),
            pl.BlockSpec((H, H), lambda p, i: (0, 0)),
            pl.BlockSpec((1, H), lambda p, i: (0, 0)),
            pl.BlockSpec((H, H), lambda p, i: (0, 0)),
            pl.BlockSpec((1, H), lambda p, i: (0, 0)),
            pl.BlockSpec((1, H), lambda p, i: (0, 0)),
            pl.BlockSpec((1, H), lambda p, i: (0, 0)),
            pl.BlockSpec((1, 1, BN), lambda p, i: (jnp.where(p == 2, i, 0), 0, 0)),
        ],
        out_specs=pl.BlockSpec((G, H), lambda p, i: (0, 0)),
        out_shape=jax.ShapeDtypeStruct((G, H), jnp.float32),
        scratch_shapes=[
            pltpu.VMEM((N, H), jnp.float32),
            pltpu.VMEM((8, H), jnp.float32),
            pltpu.VMEM((G, H), jnp.float32),
        ],
    )(eps2, h, agg, w1, b1r, w2, b2r, gammar, betar, b3)


def kernel(x, edge_index, edge_attr, batch, params):
    del edge_attr
    table = params["atom_table"]
    tab_padded = jnp.concatenate(
        [table, jnp.zeros((VOCAB_PAD - table.shape[0], H), jnp.float32)], axis=0)
    src = edge_index[0].reshape(NTILES, EPT)
    dst = edge_index[1].reshape(NTILES, NCH, ECH)
    zeros_nh = jnp.zeros((N, H), jnp.float32)

    h = _embed(x, tab_padded)
    for li, p in enumerate(params["layers"]):
        agg = _EDGE_AGG(h, src, dst, zeros_nh)
        eps2 = p["eps"].reshape(1, 1)
        args = (eps2, h, agg, p["W1"], p["b1"].reshape(1, H),
                p["W2"], p["b2"].reshape(1, H),
                p["gamma"].reshape(1, H), p["beta"].reshape(1, H))
        if li == len(params["layers"]) - 1:
            return _last_layer(*args, batch)
        h = _layer(*args)
